# baseline (speedup 1.0000x reference)
import functools

import jax
import jax.numpy as jnp
from jax import lax
from jax.experimental import pallas as pl
from jax.experimental.pallas import tpu as pltpu
from jax.experimental.pallas import tpu_sc as plsc

_IDX_CHUNK = 128


@functools.lru_cache(maxsize=None)
def _sc_gather(num_workers: int, n_chunks: int, n_rows: int, d: int):
    b_per_w = n_chunks * _IDX_CHUNK
    mesh = plsc.VectorSubcoreMesh(core_axis_name="c", subcore_axis_name="s")
    nc = 2

    @functools.partial(
        pl.kernel,
        out_type=jax.ShapeDtypeStruct((num_workers * b_per_w, d), jnp.int32),
        mesh=mesh,
        scratch_types=[
            pltpu.VMEM((n_chunks, _IDX_CHUNK), jnp.int32),
            pltpu.VMEM((b_per_w, d), jnp.int32),
            pltpu.SemaphoreType.DMA,
        ],
        compiler_params=pltpu.CompilerParams(use_tc_tiling_on_sc=False),
    )
    def gather(idx_hbm, table_hbm, out_hbm, idx_v, rows_v, sem):
        wid = lax.axis_index("s") * nc + lax.axis_index("c")
        pltpu.sync_copy(idx_hbm.at[wid], idx_v)
        copies = [
            pltpu.async_copy(
                table_hbm.at[idx_v.at[j]],
                rows_v.at[pl.ds(j * _IDX_CHUNK, _IDX_CHUNK)],
                sem,
            )
            for j in range(n_chunks)
        ]
        for c in copies:
            c.wait()
        pltpu.sync_copy(rows_v, out_hbm.at[pl.ds(wid * b_per_w, b_per_w)])

    return gather


def _tc_fused(e, fw, w1t, w2t, b2d):
    bsz, d = e.shape
    f = w2t.shape[1]
    blk = 2048

    def body(e_ref, f_ref, w1_ref, w2_ref, b_ref, o_ref):
        acc = jnp.dot(e_ref[...], w1_ref[...], preferred_element_type=jnp.float32)
        acc = acc + jnp.dot(f_ref[...], w2_ref[...], preferred_element_type=jnp.float32)
        o_ref[...] = acc + b_ref[...]

    return pl.pallas_call(
        body,
        grid=(bsz // blk,),
        in_specs=[
            pl.BlockSpec((blk, d), lambda i: (i, 0)),
            pl.BlockSpec((blk, fw.shape[1]), lambda i: (i, 0)),
            pl.BlockSpec(w1t.shape, lambda i: (0, 0)),
            pl.BlockSpec(w2t.shape, lambda i: (0, 0)),
            pl.BlockSpec((1, f), lambda i: (0, 0)),
        ],
        out_specs=pl.BlockSpec((blk, f), lambda i: (i, 0)),
        out_shape=jax.ShapeDtypeStruct((bsz, f), jnp.float32),
    )(e, fw, w1t, w2t, b2d)


def kernel(concept_embeddings, fusion_weights, emb_table, W, b):
    bsz = concept_embeddings.shape[0]
    n_rows, d = emb_table.shape
    num_workers = 32
    b_per_w = bsz // num_workers
    n_chunks = b_per_w // _IDX_CHUNK

    idx = concept_embeddings.astype(jnp.int32).reshape(
        num_workers, n_chunks, _IDX_CHUNK
    )
    table_w = lax.bitcast_convert_type(
        emb_table.astype(jnp.bfloat16).reshape(n_rows, d // 2, 2), jnp.int32
    )
    e_w = _sc_gather(num_workers, n_chunks, n_rows, d // 2)(idx, table_w)
    e = lax.bitcast_convert_type(e_w, jnp.bfloat16).reshape(bsz, d)

    w1t = W[:, :d].T.astype(jnp.bfloat16)
    w2t = W[:, d:].T
    return _tc_fused(e, fusion_weights, w1t, w2t, b.reshape(1, -1))


# speedup vs baseline: 2.0268x; 2.0268x over previous
import functools

import jax
import jax.numpy as jnp
from jax import lax
from jax.experimental import pallas as pl
from jax.experimental.pallas import tpu as pltpu
from jax.experimental.pallas import tpu_sc as plsc

_IDX_CHUNK = 128


@functools.lru_cache(maxsize=None)
def _sc_gather(num_workers: int, n_chunks: int, n_rows: int, d: int):
    b_per_w = n_chunks * _IDX_CHUNK
    mesh = plsc.VectorSubcoreMesh(core_axis_name="c", subcore_axis_name="s")
    nc = 2

    @functools.partial(
        pl.kernel,
        out_type=jax.ShapeDtypeStruct((num_workers * b_per_w, d), jnp.float32),
        mesh=mesh,
        scratch_types=[
            pltpu.VMEM((n_chunks, _IDX_CHUNK), jnp.int32),
            pltpu.VMEM((b_per_w, d), jnp.float32),
            pltpu.SemaphoreType.DMA,
        ],
    )
    def gather(idx_hbm, table_hbm, out_hbm, idx_v, rows_v, sem):
        wid = lax.axis_index("s") * nc + lax.axis_index("c")
        pltpu.sync_copy(idx_hbm.at[wid], idx_v)
        copies = [
            pltpu.async_copy(
                table_hbm.at[idx_v.at[j]],
                rows_v.at[pl.ds(j * _IDX_CHUNK, _IDX_CHUNK)],
                sem,
            )
            for j in range(n_chunks)
        ]
        for c in copies:
            c.wait()
        pltpu.sync_copy(rows_v, out_hbm.at[pl.ds(wid * b_per_w, b_per_w)])

    return gather


def _tc_partial(fw, w2t, b2d):
    bsz, d = fw.shape
    f = w2t.shape[1]
    blk = 2048

    def body(f_ref, w_ref, b_ref, o_ref):
        acc = jnp.dot(f_ref[...], w_ref[...], preferred_element_type=jnp.float32)
        o_ref[...] = (acc + b_ref[...]).astype(jnp.bfloat16)

    return pl.pallas_call(
        body,
        grid=(bsz // blk,),
        in_specs=[
            pl.BlockSpec((blk, d), lambda i: (i, 0)),
            pl.BlockSpec((d, f), lambda i: (0, 0)),
            pl.BlockSpec((1, f), lambda i: (0, 0)),
        ],
        out_specs=pl.BlockSpec((blk, f), lambda i: (i, 0)),
        out_shape=jax.ShapeDtypeStruct((bsz, f), jnp.bfloat16),
    )(fw, w2t, b2d)


def _tc_final(e, m, w1t):
    bsz, d = e.shape
    f = w1t.shape[1]
    blk = 2048

    def body(e_ref, m_ref, w1_ref, o_ref):
        acc = jnp.dot(e_ref[...], w1_ref[...], preferred_element_type=jnp.float32)
        o_ref[...] = acc + m_ref[...].astype(jnp.float32)

    return pl.pallas_call(
        body,
        grid=(bsz // blk,),
        in_specs=[
            pl.BlockSpec((blk, d), lambda i: (i, 0)),
            pl.BlockSpec((blk, f), lambda i: (i, 0)),
            pl.BlockSpec((d, f), lambda i: (0, 0)),
        ],
        out_specs=pl.BlockSpec((blk, f), lambda i: (i, 0)),
        out_shape=jax.ShapeDtypeStruct((bsz, f), jnp.float32),
    )(e, m, w1t)


def kernel(concept_embeddings, fusion_weights, emb_table, W, b):
    bsz = concept_embeddings.shape[0]
    n_rows, d = emb_table.shape
    num_workers = 32
    b_per_w = bsz // num_workers
    n_chunks = b_per_w // _IDX_CHUNK

    idx = concept_embeddings.astype(jnp.int32).reshape(
        num_workers, n_chunks, _IDX_CHUNK
    )
    e = _sc_gather(num_workers, n_chunks, n_rows, d)(idx, emb_table)

    m = _tc_partial(fusion_weights, W[:, d:].T, b.reshape(1, -1))
    return _tc_final(e, m, W[:, :d].T)


# speedup vs baseline: 2.0989x; 1.0355x over previous
import functools

import jax
import jax.numpy as jnp
from jax import lax
from jax.experimental import pallas as pl
from jax.experimental.pallas import tpu as pltpu
from jax.experimental.pallas import tpu_sc as plsc

_IDX_CHUNK = 128


@functools.lru_cache(maxsize=None)
def _sc_gather(num_workers: int, n_chunks: int, n_rows: int, d: int):
    b_per_w = n_chunks * _IDX_CHUNK
    mesh = plsc.VectorSubcoreMesh(core_axis_name="c", subcore_axis_name="s")
    nc = 2

    @functools.partial(
        pl.kernel,
        out_type=jax.ShapeDtypeStruct((num_workers * b_per_w, d), jnp.float32),
        mesh=mesh,
        scratch_types=[
            pltpu.VMEM((n_chunks, _IDX_CHUNK), jnp.int32),
            pltpu.VMEM((b_per_w, d), jnp.float32),
            pltpu.SemaphoreType.DMA((n_chunks,)),
            pltpu.SemaphoreType.DMA,
        ],
    )
    def gather(idx_hbm, table_hbm, out_hbm, idx_v, rows_v, gsems, wsem):
        wid = lax.axis_index("s") * nc + lax.axis_index("c")
        base = wid * b_per_w
        pltpu.sync_copy(idx_hbm.at[wid], idx_v)
        gathers = [
            pltpu.async_copy(
                table_hbm.at[idx_v.at[j]],
                rows_v.at[pl.ds(j * _IDX_CHUNK, _IDX_CHUNK)],
                gsems.at[j],
            )
            for j in range(n_chunks)
        ]
        writes = []
        for j in range(n_chunks):
            gathers[j].wait()
            writes.append(
                pltpu.async_copy(
                    rows_v.at[pl.ds(j * _IDX_CHUNK, _IDX_CHUNK)],
                    out_hbm.at[pl.ds(base + j * _IDX_CHUNK, _IDX_CHUNK)],
                    wsem,
                )
            )
        for c in writes:
            c.wait()

    return gather


def _tc_fused(e, fw, w1t, w2t, b2d):
    bsz, d = e.shape
    f = w2t.shape[1]
    blk = 2048

    def body(e_ref, f_ref, w1_ref, w2_ref, b_ref, o_ref):
        acc = jnp.dot(e_ref[...], w1_ref[...], preferred_element_type=jnp.float32)
        acc = acc + jnp.dot(f_ref[...], w2_ref[...], preferred_element_type=jnp.float32)
        o_ref[...] = acc + b_ref[...]

    return pl.pallas_call(
        body,
        grid=(bsz // blk,),
        in_specs=[
            pl.BlockSpec((blk, d), lambda i: (i, 0)),
            pl.BlockSpec((blk, fw.shape[1]), lambda i: (i, 0)),
            pl.BlockSpec((d, f), lambda i: (0, 0)),
            pl.BlockSpec((f, f), lambda i: (0, 0)),
            pl.BlockSpec((1, f), lambda i: (0, 0)),
        ],
        out_specs=pl.BlockSpec((blk, f), lambda i: (i, 0)),
        out_shape=jax.ShapeDtypeStruct((bsz, f), jnp.float32),
    )(e, fw, w1t, w2t, b2d)


def kernel(concept_embeddings, fusion_weights, emb_table, W, b):
    bsz = concept_embeddings.shape[0]
    n_rows, d = emb_table.shape
    num_workers = 32
    b_per_w = bsz // num_workers
    n_chunks = b_per_w // _IDX_CHUNK

    idx = concept_embeddings.astype(jnp.int32).reshape(
        num_workers, n_chunks, _IDX_CHUNK
    )
    e = _sc_gather(num_workers, n_chunks, n_rows, d)(idx, emb_table)

    w1t = W[:, :d].T
    w2t = W[:, d:].T
    return _tc_fused(e, fusion_weights, w1t, w2t, b.reshape(1, -1))


# speedup vs baseline: 2.1424x; 1.0207x over previous
import functools

import jax
import jax.numpy as jnp
from jax import lax
from jax.experimental import pallas as pl
from jax.experimental.pallas import tpu as pltpu
from jax.experimental.pallas import tpu_sc as plsc

_IDX_CHUNK = 128


@functools.lru_cache(maxsize=None)
def _sc_gather(num_workers: int, n_chunks: int, n_rows: int, d: int):
    b_per_w = n_chunks * _IDX_CHUNK
    mesh = plsc.VectorSubcoreMesh(core_axis_name="c", subcore_axis_name="s")
    nc = 2

    @functools.partial(
        pl.kernel,
        out_type=jax.ShapeDtypeStruct((num_workers * b_per_w, d), jnp.float32),
        mesh=mesh,
        scratch_types=[
            pltpu.VMEM((n_chunks, _IDX_CHUNK), jnp.int32),
            pltpu.VMEM((b_per_w, d), jnp.float32),
            pltpu.SemaphoreType.DMA,
        ],
    )
    def gather(idx_hbm, table_hbm, out_hbm, idx_v, rows_v, sem):
        wid = lax.axis_index("s") * nc + lax.axis_index("c")
        pltpu.sync_copy(idx_hbm.at[wid], idx_v)
        copies = [
            pltpu.async_copy(
                table_hbm.at[idx_v.at[j]],
                rows_v.at[pl.ds(j * _IDX_CHUNK, _IDX_CHUNK)],
                sem,
            )
            for j in range(n_chunks)
        ]
        for c in copies:
            c.wait()
        pltpu.sync_copy(rows_v, out_hbm.at[pl.ds(wid * b_per_w, b_per_w)])

    return gather


def _tc_fused(e, fw, w1, w2, b2d):
    bsz, d = e.shape
    f = w1.shape[0]
    blk = 2048
    dnums = (((1,), (1,)), ((), ()))

    def body(e_ref, f_ref, w1_ref, w2_ref, b_ref, o_ref):
        acc = lax.dot_general(
            e_ref[...], w1_ref[...], dnums, preferred_element_type=jnp.float32
        )
        acc = acc + lax.dot_general(
            f_ref[...], w2_ref[...], dnums, preferred_element_type=jnp.float32
        )
        o_ref[...] = acc + b_ref[...]

    return pl.pallas_call(
        body,
        grid=(bsz // blk,),
        in_specs=[
            pl.BlockSpec((blk, d), lambda i: (i, 0)),
            pl.BlockSpec((blk, fw.shape[1]), lambda i: (i, 0)),
            pl.BlockSpec((f, d), lambda i: (0, 0)),
            pl.BlockSpec(w2.shape, lambda i: (0, 0)),
            pl.BlockSpec((1, f), lambda i: (0, 0)),
        ],
        out_specs=pl.BlockSpec((blk, f), lambda i: (i, 0)),
        out_shape=jax.ShapeDtypeStruct((bsz, f), jnp.float32),
    )(e, fw, w1, w2, b2d)


def kernel(concept_embeddings, fusion_weights, emb_table, W, b):
    bsz = concept_embeddings.shape[0]
    n_rows, d = emb_table.shape
    num_workers = 32
    b_per_w = bsz // num_workers
    n_chunks = b_per_w // _IDX_CHUNK

    idx = concept_embeddings.astype(jnp.int32).reshape(
        num_workers, n_chunks, _IDX_CHUNK
    )
    e = _sc_gather(num_workers, n_chunks, n_rows, d)(idx, emb_table)

    return _tc_fused(e, fusion_weights, W[:, :d], W[:, d:], b.reshape(1, -1))


# speedup vs baseline: 2.1593x; 1.0079x over previous
import functools

import jax
import jax.numpy as jnp
from jax import lax
from jax.experimental import pallas as pl
from jax.experimental.pallas import tpu as pltpu
from jax.experimental.pallas import tpu_sc as plsc

_IDX_CHUNK = 128


@functools.lru_cache(maxsize=None)
def _sc_gather(num_workers: int, n_chunks: int, n_rows: int, d: int):
    b_per_w = n_chunks * _IDX_CHUNK
    mesh = plsc.VectorSubcoreMesh(core_axis_name="c", subcore_axis_name="s")
    nc = 2

    @functools.partial(
        pl.kernel,
        out_type=jax.ShapeDtypeStruct((num_workers * b_per_w, d), jnp.float32),
        mesh=mesh,
        scratch_types=[
            pltpu.VMEM((b_per_w,), jnp.int32),
            pltpu.VMEM((b_per_w, d), jnp.float32),
            pltpu.SemaphoreType.DMA,
        ],
    )
    def gather(idx_hbm, table_hbm, out_hbm, idx_v, rows_v, sem):
        wid = lax.axis_index("s") * nc + lax.axis_index("c")
        pltpu.sync_copy(idx_hbm.at[pl.ds(wid * b_per_w, b_per_w)], idx_v)
        copies = [
            pltpu.async_copy(
                table_hbm.at[idx_v.at[pl.ds(j * _IDX_CHUNK, _IDX_CHUNK)]],
                rows_v.at[pl.ds(j * _IDX_CHUNK, _IDX_CHUNK)],
                sem,
            )
            for j in range(n_chunks)
        ]
        for c in copies:
            c.wait()
        pltpu.sync_copy(rows_v, out_hbm.at[pl.ds(wid * b_per_w, b_per_w)])

    return gather


def _tc_fused(e, fw, w1, w2, b2d):
    bsz, d = e.shape
    f = w1.shape[0]
    blk = 2048
    dnums = (((1,), (1,)), ((), ()))

    def body(e_ref, f_ref, w1_ref, w2_ref, b_ref, o_ref):
        acc = lax.dot_general(
            e_ref[...], w1_ref[...], dnums, preferred_element_type=jnp.float32
        )
        acc = acc + lax.dot_general(
            f_ref[...], w2_ref[...], dnums, preferred_element_type=jnp.float32
        )
        o_ref[...] = acc + b_ref[...]

    return pl.pallas_call(
        body,
        grid=(bsz // blk,),
        in_specs=[
            pl.BlockSpec((blk, d), lambda i: (i, 0)),
            pl.BlockSpec((blk, fw.shape[1]), lambda i: (i, 0)),
            pl.BlockSpec((f, d), lambda i: (0, 0)),
            pl.BlockSpec(w2.shape, lambda i: (0, 0)),
            pl.BlockSpec((1, f), lambda i: (0, 0)),
        ],
        out_specs=pl.BlockSpec((blk, f), lambda i: (i, 0)),
        out_shape=jax.ShapeDtypeStruct((bsz, f), jnp.float32),
    )(e, fw, w1, w2, b2d)


def kernel(concept_embeddings, fusion_weights, emb_table, W, b):
    bsz = concept_embeddings.shape[0]
    n_rows, d = emb_table.shape
    num_workers = 32
    b_per_w = bsz // num_workers
    n_chunks = b_per_w // _IDX_CHUNK

    idx = concept_embeddings.astype(jnp.int32)
    e = _sc_gather(num_workers, n_chunks, n_rows, d)(idx, emb_table)

    return _tc_fused(e, fusion_weights, W[:, :d], W[:, d:], b.reshape(1, -1))


# speedup vs baseline: 2.2302x; 1.0328x over previous
import functools

import jax
import jax.numpy as jnp
from jax import lax
from jax.experimental import pallas as pl
from jax.experimental.pallas import tpu as pltpu
from jax.experimental.pallas import tpu_sc as plsc

_IDX_CHUNK = 128


@functools.lru_cache(maxsize=None)
def _sc_gather(num_workers: int, n_chunks: int, n_rows: int, d: int):
    b_per_w = n_chunks * _IDX_CHUNK
    mesh = plsc.VectorSubcoreMesh(core_axis_name="c", subcore_axis_name="s")
    nc = 2

    @functools.partial(
        pl.kernel,
        out_type=jax.ShapeDtypeStruct((num_workers * b_per_w, d), jnp.float32),
        mesh=mesh,
        scratch_types=[
            pltpu.VMEM((b_per_w,), jnp.int32),
            pltpu.VMEM((b_per_w, d), jnp.float32),
            pltpu.SemaphoreType.DMA,
        ],
    )
    def gather(idx_hbm, table_hbm, out_hbm, idx_v, rows_v, sem):
        wid = lax.axis_index("s") * nc + lax.axis_index("c")
        pltpu.sync_copy(idx_hbm.at[pl.ds(wid * b_per_w, b_per_w)], idx_v)
        copies = [
            pltpu.async_copy(
                table_hbm.at[idx_v.at[pl.ds(j * _IDX_CHUNK, _IDX_CHUNK)]],
                rows_v.at[pl.ds(j * _IDX_CHUNK, _IDX_CHUNK)],
                sem,
            )
            for j in range(n_chunks)
        ]
        for c in copies:
            c.wait()
        pltpu.sync_copy(rows_v, out_hbm.at[pl.ds(wid * b_per_w, b_per_w)])

    return gather


def _tc_fused(e, fw, w1, w2, b2d):
    bsz, d = e.shape
    f = w1.shape[0]
    blk = 4096
    dnums = (((1,), (1,)), ((), ()))

    def body(e_ref, f_ref, w1_ref, w2_ref, b_ref, o_ref):
        acc = lax.dot_general(
            e_ref[...], w1_ref[...], dnums, preferred_element_type=jnp.float32
        )
        acc = acc + lax.dot_general(
            f_ref[...], w2_ref[...], dnums, preferred_element_type=jnp.float32
        )
        o_ref[...] = acc + b_ref[...]

    return pl.pallas_call(
        body,
        grid=(bsz // blk,),
        in_specs=[
            pl.BlockSpec((blk, d), lambda i: (i, 0)),
            pl.BlockSpec((blk, fw.shape[1]), lambda i: (i, 0)),
            pl.BlockSpec((f, d), lambda i: (0, 0)),
            pl.BlockSpec(w2.shape, lambda i: (0, 0)),
            pl.BlockSpec((1, f), lambda i: (0, 0)),
        ],
        out_specs=pl.BlockSpec((blk, f), lambda i: (i, 0)),
        out_shape=jax.ShapeDtypeStruct((bsz, f), jnp.float32),
    )(e, fw, w1, w2, b2d)


def kernel(concept_embeddings, fusion_weights, emb_table, W, b):
    bsz = concept_embeddings.shape[0]
    n_rows, d = emb_table.shape
    num_workers = 32
    b_per_w = bsz // num_workers
    n_chunks = b_per_w // _IDX_CHUNK

    idx = concept_embeddings.astype(jnp.int32)
    e = _sc_gather(num_workers, n_chunks, n_rows, d)(idx, emb_table)

    return _tc_fused(e, fusion_weights, W[:, :d], W[:, d:], b.reshape(1, -1))


# speedup vs baseline: 2.3040x; 1.0331x over previous
import functools

import jax
import jax.numpy as jnp
from jax import lax
from jax.experimental import pallas as pl
from jax.experimental.pallas import tpu as pltpu
from jax.experimental.pallas import tpu_sc as plsc

_IDX_CHUNK = 128


@functools.lru_cache(maxsize=None)
def _sc_gather(num_workers: int, n_chunks: int, n_rows: int, d: int):
    b_per_w = n_chunks * _IDX_CHUNK
    mesh = plsc.VectorSubcoreMesh(core_axis_name="c", subcore_axis_name="s")
    nc = 2

    @functools.partial(
        pl.kernel,
        out_type=jax.ShapeDtypeStruct((num_workers * b_per_w, d), jnp.float32),
        mesh=mesh,
        scratch_types=[
            pltpu.VMEM((b_per_w,), jnp.int32),
            pltpu.VMEM((b_per_w, d), jnp.float32),
            pltpu.SemaphoreType.DMA,
        ],
    )
    def gather(idx_hbm, table_hbm, out_hbm, idx_v, rows_v, sem):
        wid = lax.axis_index("s") * nc + lax.axis_index("c")
        pltpu.sync_copy(idx_hbm.at[pl.ds(wid * b_per_w, b_per_w)], idx_v)
        copies = [
            pltpu.async_copy(
                table_hbm.at[idx_v.at[pl.ds(j * _IDX_CHUNK, _IDX_CHUNK)]],
                rows_v.at[pl.ds(j * _IDX_CHUNK, _IDX_CHUNK)],
                sem,
            )
            for j in range(n_chunks)
        ]
        for c in copies:
            c.wait()
        pltpu.sync_copy(rows_v, out_hbm.at[pl.ds(wid * b_per_w, b_per_w)])

    return gather


def _tc_fused(e, fw, w1, w2, b2d):
    bsz, d = e.shape
    f = w1.shape[0]
    blk = 8192
    dnums = (((1,), (1,)), ((), ()))

    def body(e_ref, f_ref, w1_ref, w2_ref, b_ref, o_ref):
        acc = lax.dot_general(
            e_ref[...], w1_ref[...], dnums, preferred_element_type=jnp.float32
        )
        acc = acc + lax.dot_general(
            f_ref[...], w2_ref[...], dnums, preferred_element_type=jnp.float32
        )
        o_ref[...] = acc + b_ref[...]

    return pl.pallas_call(
        body,
        grid=(bsz // blk,),
        in_specs=[
            pl.BlockSpec((blk, d), lambda i: (i, 0)),
            pl.BlockSpec((blk, fw.shape[1]), lambda i: (i, 0)),
            pl.BlockSpec((f, d), lambda i: (0, 0)),
            pl.BlockSpec(w2.shape, lambda i: (0, 0)),
            pl.BlockSpec((1, f), lambda i: (0, 0)),
        ],
        out_specs=pl.BlockSpec((blk, f), lambda i: (i, 0)),
        out_shape=jax.ShapeDtypeStruct((bsz, f), jnp.float32),
    )(e, fw, w1, w2, b2d)


def kernel(concept_embeddings, fusion_weights, emb_table, W, b):
    bsz = concept_embeddings.shape[0]
    n_rows, d = emb_table.shape
    num_workers = 32
    b_per_w = bsz // num_workers
    n_chunks = b_per_w // _IDX_CHUNK

    idx = concept_embeddings.astype(jnp.int32)
    e = _sc_gather(num_workers, n_chunks, n_rows, d)(idx, emb_table)

    return _tc_fused(e, fusion_weights, W[:, :d], W[:, d:], b.reshape(1, -1))
